# Initial kernel scaffold; baseline (speedup 1.0000x reference)
import jax
import jax.numpy as jnp
from jax.experimental import pallas as pl


def kernel(source, target, W1, b1, gamma, beta, W2, b2, log_temp):
    raise NotImplementedError("write your pallas kernel here")


# speedup vs baseline: 34.0687x; 34.0687x over previous
import jax
import jax.numpy as jnp
from jax import lax
from jax.experimental import pallas as pl

FDIM = 256
KNN = 16
BN = 256
BM = 1024


def _mlp(x, W1, b1, gamma, beta, W2, b2):
    h = lax.dot_general(x, W1, (((1,), (0,)), ((), ())),
                        preferred_element_type=jnp.float32) + b1
    mu = jnp.mean(h, axis=-1, keepdims=True)
    var = jnp.mean((h - mu) ** 2, axis=-1, keepdims=True)
    h = (h - mu) / jnp.sqrt(var + 1e-5) * gamma + beta
    h = jnp.maximum(h, 0.0)
    return lax.dot_general(h, W2, (((1,), (0,)), ((), ())),
                           preferred_element_type=jnp.float32) + b2


def _tgt_feat_kernel(t_ref, W1_ref, b1_ref, g_ref, be_ref, W2_ref, b2_ref,
                     tf_ref):
    tf_ref[0] = _mlp(t_ref[0], W1_ref[...], b1_ref[...], g_ref[...],
                     be_ref[...], W2_ref[...], b2_ref[...])


def _attn_kernel(s_ref, t_ref, tT_ref, tf_ref, W1_ref, b1_ref, g_ref, be_ref,
                 W2s_ref, b2s_ref, o_ref):
    s = s_ref[0]
    t = t_ref[0]
    tT = tT_ref[0]

    q = _mlp(s, W1_ref[...], b1_ref[...], g_ref[...], be_ref[...],
             W2s_ref[...], b2s_ref[...])

    ssq = jnp.sum(s * s, axis=1, keepdims=True)
    tsq = jnp.sum(tT * tT, axis=0, keepdims=True)
    cross = lax.dot_general(s, tT, (((1,), (0,)), ((), ())),
                            preferred_element_type=jnp.float32)
    d = ssq + tsq - 2.0 * cross

    inf = jnp.float32(jnp.inf)

    def peel(_, dc):
        v = jnp.min(dc, axis=1, keepdims=True)
        return jnp.where(dc == v, inf, dc)

    d = lax.fori_loop(0, KNN, peel, d, unroll=True)
    mask = d == inf

    logits = lax.dot_general(q, tf_ref[0], (((1,), (1,)), ((), ())),
                             preferred_element_type=jnp.float32)
    neg = jnp.float32(-jnp.inf)
    l = jnp.where(mask, logits, neg)
    mx = jnp.max(l, axis=1, keepdims=True)
    e = jnp.exp(l - mx)
    p = e / jnp.sum(e, axis=1, keepdims=True)
    o_ref[0] = lax.dot_general(p, t, (((1,), (0,)), ((), ())),
                               preferred_element_type=jnp.float32)


@jax.jit
def _run(source, target, W1, b1, gamma, beta, W2, b2, log_temp):
    B, N, _ = source.shape
    M = target.shape[1]
    temp = jnp.exp(log_temp[0]) * (FDIM ** 0.5)
    b1r = b1.reshape(1, FDIM)
    gr = gamma.reshape(1, FDIM)
    ber = beta.reshape(1, FDIM)
    b2r = b2.reshape(1, FDIM)
    W2s = W2 / temp
    b2s = b2r / temp
    targetT = jnp.transpose(target, (0, 2, 1))

    wspec = lambda shape: pl.BlockSpec(shape, lambda b, j: (0,) * len(shape))

    tgt_feat = pl.pallas_call(
        _tgt_feat_kernel,
        grid=(B, M // BM),
        in_specs=[
            pl.BlockSpec((1, BM, 3), lambda b, j: (b, j, 0)),
            wspec((3, FDIM)), wspec((1, FDIM)), wspec((1, FDIM)),
            wspec((1, FDIM)), wspec((FDIM, FDIM)), wspec((1, FDIM)),
        ],
        out_specs=pl.BlockSpec((1, BM, FDIM), lambda b, j: (b, j, 0)),
        out_shape=jax.ShapeDtypeStruct((B, M, FDIM), jnp.float32),
    )(target, W1, b1r, gr, ber, W2, b2r)

    out = pl.pallas_call(
        _attn_kernel,
        grid=(B, N // BN),
        in_specs=[
            pl.BlockSpec((1, BN, 3), lambda b, j: (b, j, 0)),
            pl.BlockSpec((1, M, 3), lambda b, j: (b, 0, 0)),
            pl.BlockSpec((1, 3, M), lambda b, j: (b, 0, 0)),
            pl.BlockSpec((1, M, FDIM), lambda b, j: (b, 0, 0)),
            wspec((3, FDIM)), wspec((1, FDIM)), wspec((1, FDIM)),
            wspec((1, FDIM)), wspec((FDIM, FDIM)), wspec((1, FDIM)),
        ],
        out_specs=pl.BlockSpec((1, BN, 3), lambda b, j: (b, j, 0)),
        out_shape=jax.ShapeDtypeStruct((B, N, 3), jnp.float32),
    )(source, target, targetT, tgt_feat, W1, b1r, gr, ber, W2s, b2s)
    return out


def kernel(source, target, W1, b1, gamma, beta, W2, b2, log_temp):
    return _run(source, target, W1, b1, gamma, beta, W2, b2, log_temp)


# speedup vs baseline: 36.6653x; 1.0762x over previous
import jax
import jax.numpy as jnp
from jax import lax
from jax.experimental import pallas as pl

FDIM = 256
KNN = 16
BN = 256
BM = 1024


def _mlp(x, W1, b1, gamma, beta, W2, b2):
    h = lax.dot_general(x, W1, (((1,), (0,)), ((), ())),
                        preferred_element_type=jnp.float32) + b1
    mu = jnp.mean(h, axis=-1, keepdims=True)
    var = jnp.mean((h - mu) ** 2, axis=-1, keepdims=True)
    h = (h - mu) / jnp.sqrt(var + 1e-5) * gamma + beta
    h = jnp.maximum(h, 0.0)
    return lax.dot_general(h, W2, (((1,), (0,)), ((), ())),
                           preferred_element_type=jnp.float32) + b2


def _tgt_feat_kernel(t_ref, tT_ref, W1_ref, b1_ref, g_ref, be_ref, W2_ref,
                     b2_ref, tf_ref, tsq_ref):
    tf_ref[0] = _mlp(t_ref[0], W1_ref[...], b1_ref[...], g_ref[...],
                     be_ref[...], W2_ref[...], b2_ref[...])
    tT = tT_ref[0]
    tsq_ref[0] = jnp.sum(tT * tT, axis=0, keepdims=True)


def _attn_kernel(s_ref, t_ref, tTs_ref, tsq_ref, tf_ref, W1_ref, b1_ref,
                 g_ref, be_ref, W2s_ref, b2s_ref, o_ref):
    s = s_ref[0]
    t = t_ref[0]
    tTs = tTs_ref[0]
    tf = tf_ref[0]
    M = t.shape[0]
    Q = M // 4

    q = _mlp(s, W1_ref[...], b1_ref[...], g_ref[...], be_ref[...],
             W2s_ref[...], b2s_ref[...])

    ssq = jnp.sum(s * s, axis=1, keepdims=True)
    tsq = tsq_ref[0]
    c2 = lax.dot_general(s, tTs, (((1,), (0,)), ((), ())),
                         preferred_element_type=jnp.float32)
    d = (ssq + tsq) + c2

    a = d[:, 0 * Q:1 * Q]
    b = d[:, 1 * Q:2 * Q]
    c = d[:, 2 * Q:3 * Q]
    e = d[:, 3 * Q:4 * Q]
    l1 = jnp.minimum(a, b)
    h1 = jnp.maximum(a, b)
    l2 = jnp.minimum(c, e)
    h2 = jnp.maximum(c, e)
    s1 = jnp.minimum(l1, l2)
    t1 = jnp.maximum(l1, l2)
    s4 = jnp.maximum(h1, h2)
    t2 = jnp.minimum(h1, h2)
    s2 = jnp.minimum(t1, t2)
    s3 = jnp.maximum(t1, t2)

    inf = jnp.float32(jnp.inf)

    def peel(_, carry):
        S1, S2, S3, S4, _ = carry
        v = jnp.min(S1, axis=1, keepdims=True)
        sel = S1 == v
        return (jnp.where(sel, S2, S1), jnp.where(sel, S3, S2),
                jnp.where(sel, S4, S3), jnp.where(sel, inf, S4), v)

    tau0 = jnp.zeros_like(ssq)
    _, _, _, _, tau = lax.fori_loop(0, KNN, peel, (s1, s2, s3, s4, tau0),
                                    unroll=True)

    mask = d <= tau
    logits = lax.dot_general(q, tf, (((1,), (1,)), ((), ())),
                             preferred_element_type=jnp.float32)
    neg = jnp.float32(-jnp.inf)
    l = jnp.where(mask, logits, neg)
    mx = jnp.max(l, axis=1, keepdims=True)
    ex = jnp.exp(l - mx)
    ssum = jnp.sum(ex, axis=1, keepdims=True)
    acc = lax.dot_general(ex, t, (((1,), (0,)), ((), ())),
                          preferred_element_type=jnp.float32)
    o_ref[0] = acc / ssum


@jax.jit
def _run(source, target, W1, b1, gamma, beta, W2, b2, log_temp):
    B, N, _ = source.shape
    M = target.shape[1]
    temp = jnp.exp(log_temp[0]) * (FDIM ** 0.5)
    b1r = b1.reshape(1, FDIM)
    gr = gamma.reshape(1, FDIM)
    ber = beta.reshape(1, FDIM)
    b2r = b2.reshape(1, FDIM)
    W2s = W2 / temp
    b2s = b2r / temp
    targetT = jnp.transpose(target, (0, 2, 1))
    targetTs = -2.0 * targetT

    wspec = lambda shape: pl.BlockSpec(shape, lambda b, j: (0,) * len(shape))

    tgt_feat, tgt_sq = pl.pallas_call(
        _tgt_feat_kernel,
        grid=(B, M // BM),
        in_specs=[
            pl.BlockSpec((1, BM, 3), lambda b, j: (b, j, 0)),
            pl.BlockSpec((1, 3, BM), lambda b, j: (b, 0, j)),
            wspec((3, FDIM)), wspec((1, FDIM)), wspec((1, FDIM)),
            wspec((1, FDIM)), wspec((FDIM, FDIM)), wspec((1, FDIM)),
        ],
        out_specs=[
            pl.BlockSpec((1, BM, FDIM), lambda b, j: (b, j, 0)),
            pl.BlockSpec((1, 1, BM), lambda b, j: (b, 0, j)),
        ],
        out_shape=[
            jax.ShapeDtypeStruct((B, M, FDIM), jnp.float32),
            jax.ShapeDtypeStruct((B, 1, M), jnp.float32),
        ],
    )(target, targetT, W1, b1r, gr, ber, W2, b2r)

    out = pl.pallas_call(
        _attn_kernel,
        grid=(B, N // BN),
        in_specs=[
            pl.BlockSpec((1, BN, 3), lambda b, j: (b, j, 0)),
            pl.BlockSpec((1, M, 3), lambda b, j: (b, 0, 0)),
            pl.BlockSpec((1, 3, M), lambda b, j: (b, 0, 0)),
            pl.BlockSpec((1, 1, M), lambda b, j: (b, 0, 0)),
            pl.BlockSpec((1, M, FDIM), lambda b, j: (b, 0, 0)),
            wspec((3, FDIM)), wspec((1, FDIM)), wspec((1, FDIM)),
            wspec((1, FDIM)), wspec((FDIM, FDIM)), wspec((1, FDIM)),
        ],
        out_specs=pl.BlockSpec((1, BN, 3), lambda b, j: (b, j, 0)),
        out_shape=jax.ShapeDtypeStruct((B, N, 3), jnp.float32),
    )(source, target, targetTs, tgt_sq, tgt_feat, W1, b1r, gr, ber, W2s, b2s)
    return out


def kernel(source, target, W1, b1, gamma, beta, W2, b2, log_temp):
    return _run(source, target, W1, b1, gamma, beta, W2, b2, log_temp)


# speedup vs baseline: 37.5571x; 1.0243x over previous
import jax
import jax.numpy as jnp
from jax import lax
from jax.experimental import pallas as pl

FDIM = 256
KNN = 16
BN = 512
BM = 1024


def _mlp(x, W1, b1, gamma, beta, W2, b2):
    h = lax.dot_general(x, W1, (((1,), (0,)), ((), ())),
                        preferred_element_type=jnp.float32) + b1
    mu = jnp.mean(h, axis=-1, keepdims=True)
    var = jnp.mean((h - mu) ** 2, axis=-1, keepdims=True)
    h = (h - mu) / jnp.sqrt(var + 1e-5) * gamma + beta
    h = jnp.maximum(h, 0.0)
    return lax.dot_general(h, W2, (((1,), (0,)), ((), ())),
                           preferred_element_type=jnp.float32) + b2


def _tgt_feat_kernel(t_ref, tT_ref, W1_ref, b1_ref, g_ref, be_ref, W2_ref,
                     b2_ref, tf_ref, tsq_ref):
    tf_ref[0] = _mlp(t_ref[0], W1_ref[...], b1_ref[...], g_ref[...],
                     be_ref[...], W2_ref[...], b2_ref[...])
    tT = tT_ref[0]
    tsq_ref[0] = jnp.sum(tT * tT, axis=0, keepdims=True)


def _attn_kernel(s_ref, t_ref, tTs_ref, tsq_ref, tf_ref, W1_ref, b1_ref,
                 g_ref, be_ref, W2s_ref, b2s_ref, o_ref):
    s = s_ref[0]
    t = t_ref[0]
    tTs = tTs_ref[0]
    tf = tf_ref[0]
    M = t.shape[0]
    Q = M // 4

    q = _mlp(s, W1_ref[...], b1_ref[...], g_ref[...], be_ref[...],
             W2s_ref[...], b2s_ref[...])

    ssq = jnp.sum(s * s, axis=1, keepdims=True)
    tsq = tsq_ref[0]
    c2 = lax.dot_general(s, tTs, (((1,), (0,)), ((), ())),
                         preferred_element_type=jnp.float32)
    d = (ssq + tsq) + c2

    a = d[:, 0 * Q:1 * Q]
    b = d[:, 1 * Q:2 * Q]
    c = d[:, 2 * Q:3 * Q]
    e = d[:, 3 * Q:4 * Q]
    l1 = jnp.minimum(a, b)
    h1 = jnp.maximum(a, b)
    l2 = jnp.minimum(c, e)
    h2 = jnp.maximum(c, e)
    s1 = jnp.minimum(l1, l2)
    t1 = jnp.maximum(l1, l2)
    s4 = jnp.maximum(h1, h2)
    t2 = jnp.minimum(h1, h2)
    s2 = jnp.minimum(t1, t2)
    s3 = jnp.maximum(t1, t2)

    inf = jnp.float32(jnp.inf)

    def peel(_, carry):
        S1, S2, S3, S4, _ = carry
        v = jnp.min(S1, axis=1, keepdims=True)
        sel = S1 == v
        return (jnp.where(sel, S2, S1), jnp.where(sel, S3, S2),
                jnp.where(sel, S4, S3), jnp.where(sel, inf, S4), v)

    tau0 = jnp.zeros_like(ssq)
    _, _, _, _, tau = lax.fori_loop(0, KNN, peel, (s1, s2, s3, s4, tau0),
                                    unroll=True)

    mask = d <= tau
    logits = lax.dot_general(q, tf, (((1,), (1,)), ((), ())),
                             preferred_element_type=jnp.float32)
    neg = jnp.float32(-jnp.inf)
    l = jnp.where(mask, logits, neg)
    mx = jnp.max(l, axis=1, keepdims=True)
    ex = jnp.exp(l - mx)
    ssum = jnp.sum(ex, axis=1, keepdims=True)
    acc = lax.dot_general(ex, t, (((1,), (0,)), ((), ())),
                          preferred_element_type=jnp.float32)
    o_ref[0] = acc / ssum


@jax.jit
def _run(source, target, W1, b1, gamma, beta, W2, b2, log_temp):
    B, N, _ = source.shape
    M = target.shape[1]
    temp = jnp.exp(log_temp[0]) * (FDIM ** 0.5)
    b1r = b1.reshape(1, FDIM)
    gr = gamma.reshape(1, FDIM)
    ber = beta.reshape(1, FDIM)
    b2r = b2.reshape(1, FDIM)
    W2s = W2 / temp
    b2s = b2r / temp
    targetT = jnp.transpose(target, (0, 2, 1))
    targetTs = -2.0 * targetT

    wspec = lambda shape: pl.BlockSpec(shape, lambda b, j: (0,) * len(shape))

    tgt_feat, tgt_sq = pl.pallas_call(
        _tgt_feat_kernel,
        grid=(B, M // BM),
        in_specs=[
            pl.BlockSpec((1, BM, 3), lambda b, j: (b, j, 0)),
            pl.BlockSpec((1, 3, BM), lambda b, j: (b, 0, j)),
            wspec((3, FDIM)), wspec((1, FDIM)), wspec((1, FDIM)),
            wspec((1, FDIM)), wspec((FDIM, FDIM)), wspec((1, FDIM)),
        ],
        out_specs=[
            pl.BlockSpec((1, BM, FDIM), lambda b, j: (b, j, 0)),
            pl.BlockSpec((1, 1, BM), lambda b, j: (b, 0, j)),
        ],
        out_shape=[
            jax.ShapeDtypeStruct((B, M, FDIM), jnp.float32),
            jax.ShapeDtypeStruct((B, 1, M), jnp.float32),
        ],
    )(target, targetT, W1, b1r, gr, ber, W2, b2r)

    out = pl.pallas_call(
        _attn_kernel,
        grid=(B, N // BN),
        in_specs=[
            pl.BlockSpec((1, BN, 3), lambda b, j: (b, j, 0)),
            pl.BlockSpec((1, M, 3), lambda b, j: (b, 0, 0)),
            pl.BlockSpec((1, 3, M), lambda b, j: (b, 0, 0)),
            pl.BlockSpec((1, 1, M), lambda b, j: (b, 0, 0)),
            pl.BlockSpec((1, M, FDIM), lambda b, j: (b, 0, 0)),
            wspec((3, FDIM)), wspec((1, FDIM)), wspec((1, FDIM)),
            wspec((1, FDIM)), wspec((FDIM, FDIM)), wspec((1, FDIM)),
        ],
        out_specs=pl.BlockSpec((1, BN, 3), lambda b, j: (b, j, 0)),
        out_shape=jax.ShapeDtypeStruct((B, N, 3), jnp.float32),
    )(source, target, targetTs, tgt_sq, tgt_feat, W1, b1r, gr, ber, W2s, b2s)
    return out


def kernel(source, target, W1, b1, gamma, beta, W2, b2, log_temp):
    return _run(source, target, W1, b1, gamma, beta, W2, b2, log_temp)


# speedup vs baseline: 37.5859x; 1.0008x over previous
import jax
import jax.numpy as jnp
from jax import lax
from jax.experimental import pallas as pl

FDIM = 256
KNN = 16
BN = 512
BM = 1024


def _mlp(x, W1, b1, gamma, beta, W2, b2):
    h = lax.dot_general(x, W1, (((1,), (0,)), ((), ())),
                        preferred_element_type=jnp.float32) + b1
    mu = jnp.mean(h, axis=-1, keepdims=True)
    var = jnp.mean((h - mu) ** 2, axis=-1, keepdims=True)
    h = (h - mu) / jnp.sqrt(var + 1e-5) * gamma + beta
    h = jnp.maximum(h, 0.0)
    return lax.dot_general(h, W2, (((1,), (0,)), ((), ())),
                           preferred_element_type=jnp.float32) + b2


def _tgt_feat_kernel(t_ref, tT_ref, W1_ref, b1_ref, g_ref, be_ref, W2_ref,
                     b2_ref, tf_ref, tsq_ref):
    tf_ref[0] = _mlp(t_ref[0], W1_ref[...], b1_ref[...], g_ref[...],
                     be_ref[...], W2_ref[...], b2_ref[...])
    tT = tT_ref[0]
    tsq_ref[0] = jnp.sum(tT * tT, axis=0, keepdims=True)


def _attn_kernel(s_ref, t_ref, tTs_ref, tsq_ref, tf_ref, W1_ref, b1_ref,
                 g_ref, be_ref, W2s_ref, b2s_ref, o_ref):
    s = s_ref[0]
    t = t_ref[0]
    tTs = tTs_ref[0]
    tf = tf_ref[0]
    M = t.shape[0]
    Q = M // 4

    q = _mlp(s, W1_ref[...], b1_ref[...], g_ref[...], be_ref[...],
             W2s_ref[...], b2s_ref[...])

    ssq = jnp.sum(s * s, axis=1, keepdims=True)
    tsq = tsq_ref[0]
    c2 = lax.dot_general(s, tTs, (((1,), (0,)), ((), ())),
                         preferred_element_type=jnp.float32)
    d = (ssq + tsq) + c2

    a = d[:, 0 * Q:1 * Q]
    b = d[:, 1 * Q:2 * Q]
    c = d[:, 2 * Q:3 * Q]
    e = d[:, 3 * Q:4 * Q]
    l1 = jnp.minimum(a, b)
    h1 = jnp.maximum(a, b)
    l2 = jnp.minimum(c, e)
    h2 = jnp.maximum(c, e)
    s1 = jnp.minimum(l1, l2)
    t1 = jnp.maximum(l1, l2)
    s4 = jnp.maximum(h1, h2)
    t2 = jnp.minimum(h1, h2)
    s2 = jnp.minimum(t1, t2)
    s3 = jnp.maximum(t1, t2)

    inf = jnp.float32(jnp.inf)

    def peel(_, carry):
        S1, S2, S3, S4 = carry
        v = jnp.min(S1, axis=1, keepdims=True)
        sel = S1 == v
        return (jnp.where(sel, S2, S1), jnp.where(sel, S3, S2),
                jnp.where(sel, S4, S3), jnp.where(sel, inf, S4))

    S1, _, _, _ = lax.fori_loop(0, KNN - 1, peel, (s1, s2, s3, s4),
                                unroll=True)
    tau = jnp.min(S1, axis=1, keepdims=True)

    mask = d <= tau
    logits = lax.dot_general(q, tf, (((1,), (1,)), ((), ())),
                             preferred_element_type=jnp.float32)
    neg = jnp.float32(-jnp.inf)
    l = jnp.where(mask, logits, neg)
    mx = jnp.max(l, axis=1, keepdims=True)
    ex = jnp.exp(l - mx)
    ssum = jnp.sum(ex, axis=1, keepdims=True)
    acc = lax.dot_general(ex, t, (((1,), (0,)), ((), ())),
                          preferred_element_type=jnp.float32)
    o_ref[0] = acc / ssum


@jax.jit
def _run(source, target, W1, b1, gamma, beta, W2, b2, log_temp):
    B, N, _ = source.shape
    M = target.shape[1]
    temp = jnp.exp(log_temp[0]) * (FDIM ** 0.5)
    b1r = b1.reshape(1, FDIM)
    gr = gamma.reshape(1, FDIM)
    ber = beta.reshape(1, FDIM)
    b2r = b2.reshape(1, FDIM)
    W2s = W2 / temp
    b2s = b2r / temp
    targetT = jnp.transpose(target, (0, 2, 1))
    targetTs = -2.0 * targetT

    wspec = lambda shape: pl.BlockSpec(shape, lambda b, j: (0,) * len(shape))

    tgt_feat, tgt_sq = pl.pallas_call(
        _tgt_feat_kernel,
        grid=(B, M // BM),
        in_specs=[
            pl.BlockSpec((1, BM, 3), lambda b, j: (b, j, 0)),
            pl.BlockSpec((1, 3, BM), lambda b, j: (b, 0, j)),
            wspec((3, FDIM)), wspec((1, FDIM)), wspec((1, FDIM)),
            wspec((1, FDIM)), wspec((FDIM, FDIM)), wspec((1, FDIM)),
        ],
        out_specs=[
            pl.BlockSpec((1, BM, FDIM), lambda b, j: (b, j, 0)),
            pl.BlockSpec((1, 1, BM), lambda b, j: (b, 0, j)),
        ],
        out_shape=[
            jax.ShapeDtypeStruct((B, M, FDIM), jnp.float32),
            jax.ShapeDtypeStruct((B, 1, M), jnp.float32),
        ],
    )(target, targetT, W1, b1r, gr, ber, W2, b2r)

    out = pl.pallas_call(
        _attn_kernel,
        grid=(B, N // BN),
        in_specs=[
            pl.BlockSpec((1, BN, 3), lambda b, j: (b, j, 0)),
            pl.BlockSpec((1, M, 3), lambda b, j: (b, 0, 0)),
            pl.BlockSpec((1, 3, M), lambda b, j: (b, 0, 0)),
            pl.BlockSpec((1, 1, M), lambda b, j: (b, 0, 0)),
            pl.BlockSpec((1, M, FDIM), lambda b, j: (b, 0, 0)),
            wspec((3, FDIM)), wspec((1, FDIM)), wspec((1, FDIM)),
            wspec((1, FDIM)), wspec((FDIM, FDIM)), wspec((1, FDIM)),
        ],
        out_specs=pl.BlockSpec((1, BN, 3), lambda b, j: (b, j, 0)),
        out_shape=jax.ShapeDtypeStruct((B, N, 3), jnp.float32),
    )(source, target, targetTs, tgt_sq, tgt_feat, W1, b1r, gr, ber, W2s, b2s)
    return out


def kernel(source, target, W1, b1, gamma, beta, W2, b2, log_temp):
    return _run(source, target, W1, b1, gamma, beta, W2, b2, log_temp)


# speedup vs baseline: 38.1149x; 1.0141x over previous
import jax
import jax.numpy as jnp
from jax import lax
from jax.experimental import pallas as pl
from jax.experimental.pallas import tpu as pltpu

FDIM = 256
KNN = 16
BN = 512


def _mlp(x, W1, b1, gamma, beta, W2, b2):
    h = lax.dot_general(x, W1, (((1,), (0,)), ((), ())),
                        preferred_element_type=jnp.float32) + b1
    mu = jnp.mean(h, axis=-1, keepdims=True)
    var = jnp.mean((h - mu) ** 2, axis=-1, keepdims=True)
    h = (h - mu) / jnp.sqrt(var + 1e-5) * gamma + beta
    h = jnp.maximum(h, 0.0)
    return lax.dot_general(h, W2, (((1,), (0,)), ((), ())),
                           preferred_element_type=jnp.float32) + b2


def _fused_kernel(s_ref, t_ref, tTs_ref, W1_ref, b1_ref, g_ref, be_ref,
                  W2_ref, b2_ref, W2s_ref, b2s_ref, o_ref, tf_s, tsq_s):
    j = pl.program_id(1)
    t = t_ref[0]
    tTs = tTs_ref[0]
    M = t.shape[0]
    Q = M // 4

    @pl.when(j == 0)
    def _build_target_features():
        tf_s[...] = _mlp(t, W1_ref[...], b1_ref[...], g_ref[...],
                         be_ref[...], W2_ref[...], b2_ref[...])
        tsq_s[...] = jnp.sum(tTs * tTs, axis=0, keepdims=True) * 0.25

    @pl.when(j > 0)
    def _attend():
        s = s_ref[0]

        q = _mlp(s, W1_ref[...], b1_ref[...], g_ref[...], be_ref[...],
                 W2s_ref[...], b2s_ref[...])

        ssq = jnp.sum(s * s, axis=1, keepdims=True)
        tsq = tsq_s[...]
        c2 = lax.dot_general(s, tTs, (((1,), (0,)), ((), ())),
                             preferred_element_type=jnp.float32)
        d = (ssq + tsq) + c2

        a = d[:, 0 * Q:1 * Q]
        b = d[:, 1 * Q:2 * Q]
        c = d[:, 2 * Q:3 * Q]
        e = d[:, 3 * Q:4 * Q]
        l1 = jnp.minimum(a, b)
        h1 = jnp.maximum(a, b)
        l2 = jnp.minimum(c, e)
        h2 = jnp.maximum(c, e)
        s1 = jnp.minimum(l1, l2)
        t1 = jnp.maximum(l1, l2)
        s4 = jnp.maximum(h1, h2)
        t2 = jnp.minimum(h1, h2)
        s2 = jnp.minimum(t1, t2)
        s3 = jnp.maximum(t1, t2)

        inf = jnp.float32(jnp.inf)

        def peel(_, carry):
            S1, S2, S3, S4 = carry
            v = jnp.min(S1, axis=1, keepdims=True)
            sel = S1 == v
            return (jnp.where(sel, S2, S1), jnp.where(sel, S3, S2),
                    jnp.where(sel, S4, S3), jnp.where(sel, inf, S4))

        S1, _, _, _ = lax.fori_loop(0, KNN - 1, peel, (s1, s2, s3, s4),
                                    unroll=True)
        tau = jnp.min(S1, axis=1, keepdims=True)

        mask = d <= tau
        logits = lax.dot_general(q, tf_s[...], (((1,), (1,)), ((), ())),
                                 preferred_element_type=jnp.float32)
        neg = jnp.float32(-jnp.inf)
        l = jnp.where(mask, logits, neg)
        mx = jnp.max(l, axis=1, keepdims=True)
        ex = jnp.exp(l - mx)
        ssum = jnp.sum(ex, axis=1, keepdims=True)
        acc = lax.dot_general(ex, t, (((1,), (0,)), ((), ())),
                              preferred_element_type=jnp.float32)
        o_ref[0] = acc / ssum


@jax.jit
def _run(source, target, W1, b1, gamma, beta, W2, b2, log_temp):
    B, N, _ = source.shape
    M = target.shape[1]
    temp = jnp.exp(log_temp[0]) * (FDIM ** 0.5)
    b1r = b1.reshape(1, FDIM)
    gr = gamma.reshape(1, FDIM)
    ber = beta.reshape(1, FDIM)
    b2r = b2.reshape(1, FDIM)
    W2s = W2 / temp
    b2s = b2r / temp
    targetT = jnp.transpose(target, (0, 2, 1))
    targetTs = -2.0 * targetT

    wspec = lambda shape: pl.BlockSpec(shape, lambda b, j: (0,) * len(shape))

    out = pl.pallas_call(
        _fused_kernel,
        grid=(B, 1 + N // BN),
        in_specs=[
            pl.BlockSpec((1, BN, 3),
                         lambda b, j: (b, jnp.maximum(j - 1, 0), 0)),
            pl.BlockSpec((1, M, 3), lambda b, j: (b, 0, 0)),
            pl.BlockSpec((1, 3, M), lambda b, j: (b, 0, 0)),
            wspec((3, FDIM)), wspec((1, FDIM)), wspec((1, FDIM)),
            wspec((1, FDIM)), wspec((FDIM, FDIM)), wspec((1, FDIM)),
            wspec((FDIM, FDIM)), wspec((1, FDIM)),
        ],
        out_specs=pl.BlockSpec((1, BN, 3),
                               lambda b, j: (b, jnp.maximum(j - 1, 0), 0)),
        out_shape=jax.ShapeDtypeStruct((B, N, 3), jnp.float32),
        scratch_shapes=[
            pltpu.VMEM((M, FDIM), jnp.float32),
            pltpu.VMEM((1, M), jnp.float32),
        ],
    )(source, target, targetTs, W1, b1r, gr, ber, W2, b2r, W2s, b2s)
    return out


def kernel(source, target, W1, b1, gamma, beta, W2, b2, log_temp):
    return _run(source, target, W1, b1, gamma, beta, W2, b2, log_temp)
